# baseline (speedup 1.0000x reference)
import functools

import jax
import jax.numpy as jnp
from jax import lax
from jax.experimental import pallas as pl
from jax.experimental.pallas import tpu as pltpu
from jax.experimental.pallas import tpu_sc as plsc

_ROWS_C0 = 160
_ROWS_C1 = 96


def _gather_call(table, idx, B, D, NC, NS):
    r0, r1 = _ROWS_C0, _ROWS_C1
    assert NC == 2 and NS * (r0 + r1) == B
    mesh = plsc.VectorSubcoreMesh(core_axis_name="c", subcore_axis_name="s")
    rmax = max(r0, r1)

    @functools.partial(
        pl.kernel,
        mesh=mesh,
        out_type=jax.ShapeDtypeStruct((B, D), jnp.float32),
        scratch_types=[
            pltpu.VMEM((rmax,), jnp.int32),
            pltpu.VMEM((rmax, D), jnp.float32),
            pltpu.SemaphoreType.DMA,
        ],
    )
    def gather_kernel(table_hbm, idx_hbm, out_hbm, idx_v, rows_v, sem):
        c = lax.axis_index("c")
        s = lax.axis_index("s")

        def do(base, n):
            pltpu.sync_copy(idx_hbm.at[pl.ds(base, n)], idx_v.at[pl.ds(0, n)])
            pltpu.async_copy(
                table_hbm.at[idx_v.at[pl.ds(0, n)]],
                rows_v.at[pl.ds(0, n)],
                sem,
            ).wait()
            pltpu.sync_copy(rows_v.at[pl.ds(0, n)], out_hbm.at[pl.ds(base, n)])

        @pl.when(c == 0)
        def _():
            do(s * r0, r0)

        @pl.when(c != 0)
        def _():
            do(NS * r0 + s * r1, r1)

    return gather_kernel(table, idx)


def kernel(adj_tensor, node_labels, emb_table):
    del adj_tensor
    B, = node_labels.shape
    D = emb_table.shape[1]
    info = plsc.get_sparse_core_info()
    idx = node_labels.astype(jnp.int32)
    return _gather_call(emb_table, idx, B, D, info.num_cores, info.num_subcores)


# speedup vs baseline: 1.0308x; 1.0308x over previous
import functools

import jax
import jax.numpy as jnp
from jax import lax
from jax.experimental import pallas as pl
from jax.experimental.pallas import tpu as pltpu
from jax.experimental.pallas import tpu_sc as plsc

_ROWS_C0 = 96
_ROWS_C1 = 160


def _gather_call(table, idx, B, D, NC, NS):
    r0, r1 = _ROWS_C0, _ROWS_C1
    assert NC == 2 and NS * (r0 + r1) == B
    mesh = plsc.VectorSubcoreMesh(core_axis_name="c", subcore_axis_name="s")
    rmax = max(r0, r1)

    @functools.partial(
        pl.kernel,
        mesh=mesh,
        out_type=jax.ShapeDtypeStruct((B, D), jnp.float32),
        scratch_types=[
            pltpu.VMEM((rmax,), jnp.int32),
            pltpu.VMEM((rmax, D), jnp.float32),
            pltpu.SemaphoreType.DMA,
        ],
    )
    def gather_kernel(table_hbm, idx_hbm, out_hbm, idx_v, rows_v, sem):
        c = lax.axis_index("c")
        s = lax.axis_index("s")

        def do(base, n):
            pltpu.sync_copy(idx_hbm.at[pl.ds(base, n)], idx_v.at[pl.ds(0, n)])
            pltpu.async_copy(
                table_hbm.at[idx_v.at[pl.ds(0, n)]],
                rows_v.at[pl.ds(0, n)],
                sem,
            ).wait()
            pltpu.sync_copy(rows_v.at[pl.ds(0, n)], out_hbm.at[pl.ds(base, n)])

        @pl.when(c == 0)
        def _():
            do(s * r0, r0)

        @pl.when(c != 0)
        def _():
            do(NS * r0 + s * r1, r1)

    return gather_kernel(table, idx)


def kernel(adj_tensor, node_labels, emb_table):
    del adj_tensor
    B, = node_labels.shape
    D = emb_table.shape[1]
    info = plsc.get_sparse_core_info()
    idx = node_labels.astype(jnp.int32)
    return _gather_call(emb_table, idx, B, D, info.num_cores, info.num_subcores)
